# baseline (speedup 1.0000x reference)
import functools

import jax
import jax.numpy as jnp
from jax import lax
from jax.experimental import pallas as pl
from jax.experimental.pallas import tpu as pltpu
from jax.experimental.pallas import tpu_sc as plsc

INTERPRET = False

H = 64
NC, NS = 2, 16
NW = NC * NS


def _elu(h):
    return jnp.where(h > 0.0, h, jnp.exp(h) - 1.0)


def _proj_body(x_ref, w_ref, tb_ref, xr_ref, xc_ref):
    y = jnp.dot(x_ref[...], w_ref[...], preferred_element_type=jnp.float32)
    xr_ref[...] = y[:, :H] + tb_ref[:, :H]
    xc_ref[...] = y[:, H:]


def _proj(x, Wrc, Tb):
    N, F = x.shape
    BN = 2000
    return pl.pallas_call(
        _proj_body,
        grid=(N // BN,),
        in_specs=[
            pl.BlockSpec((BN, F), lambda i: (i, 0)),
            pl.BlockSpec((F, 2 * H), lambda i: (0, 0)),
            pl.BlockSpec((BN, 2 * H), lambda i: (i, 0)),
        ],
        out_specs=[
            pl.BlockSpec((BN, H), lambda i: (i, 0)),
            pl.BlockSpec((BN, H), lambda i: (i, 0)),
        ],
        out_shape=[
            jax.ShapeDtypeStruct((N, H), jnp.float32),
            jax.ShapeDtypeStruct((N, H), jnp.float32),
        ],
        interpret=INTERPRET,
    )(x, Wrc, Tb)


def _tmat_body(u_ref, w_ref, b_ref, t_ref):
    t_ref[...] = jnp.dot(u_ref[...], w_ref[...],
                         preferred_element_type=jnp.float32) + b_ref[...]


def _tmat(u, Wut, but):
    G_, GF_ = u.shape
    return pl.pallas_call(
        _tmat_body,
        grid=(1,),
        in_specs=[
            pl.BlockSpec((G_, GF_), lambda i: (0, 0)),
            pl.BlockSpec((GF_, 2 * H), lambda i: (0, 0)),
            pl.BlockSpec((1, 2 * H), lambda i: (0, 0)),
        ],
        out_specs=pl.BlockSpec((G_, 2 * H), lambda i: (0, 0)),
        out_shape=jax.ShapeDtypeStruct((G_, 2 * H), jnp.float32),
        interpret=INTERPRET,
    )(u, Wut, but)


def _edge_body(ea_ref, g0_ref, g1_ref, we_ref, wh_ref, bh_ref, wo_ref,
               bo_ref, out_ref):
    h = _elu(jnp.dot(ea_ref[...], we_ref[...],
                     preferred_element_type=jnp.float32)
             + g0_ref[...] + g1_ref[...])
    h = _elu(jnp.dot(h, wh_ref[...],
                     preferred_element_type=jnp.float32) + bh_ref[...])
    h = _elu(jnp.dot(h, wo_ref[...],
                     preferred_element_type=jnp.float32) + bo_ref[...])
    out_ref[...] = h


def _edge_body_prev(ea_ref, ep_ref, g0_ref, g1_ref, we_ref, wh_ref, bh_ref,
                    wo_ref, bo_ref, out_ref):
    h = _elu(jnp.dot(ea_ref[...] + ep_ref[...], we_ref[...],
                     preferred_element_type=jnp.float32)
             + g0_ref[...] + g1_ref[...])
    h = _elu(jnp.dot(h, wh_ref[...],
                     preferred_element_type=jnp.float32) + bh_ref[...])
    h = _elu(jnp.dot(h, wo_ref[...],
                     preferred_element_type=jnp.float32) + bo_ref[...])
    out_ref[...] = h


def _edge_mlp(ea, G0, G1, We, Wh, bh, Wo, bo, ea_prev=None):
    E, EFd = ea.shape
    OUT = Wo.shape[1]
    BE = 2000
    wspecs = [
        pl.BlockSpec((EFd, H), lambda i: (0, 0)),
        pl.BlockSpec((H, H), lambda i: (0, 0)),
        pl.BlockSpec((1, H), lambda i: (0, 0)),
        pl.BlockSpec((H, OUT), lambda i: (0, 0)),
        pl.BlockSpec((1, OUT), lambda i: (0, 0)),
    ]
    dspec = pl.BlockSpec((BE, EFd), lambda i: (i, 0))
    gspec = pl.BlockSpec((BE, H), lambda i: (i, 0))
    if ea_prev is None:
        body, args, ins = _edge_body, (ea, G0, G1), [dspec, gspec, gspec] + wspecs
    else:
        body, args, ins = (_edge_body_prev, (ea, ea_prev, G0, G1),
                           [dspec, dspec, gspec, gspec] + wspecs)
    return pl.pallas_call(
        body,
        grid=(E // BE,),
        in_specs=ins,
        out_specs=pl.BlockSpec((BE, OUT), lambda i: (i, 0)),
        out_shape=jax.ShapeDtypeStruct((E, OUT), jnp.float32),
        interpret=INTERPRET,
    )(*args, We, Wh, bh, Wo, bo)


def _node_body(x_ref, ns0_ref, ns1_ref, c0_ref, c1_ref, tb_ref, ax_ref,
               aa_ref, ah_ref, bh_ref, ao_ref, bo_ref, *rest):
    if len(rest) == 3:
        xs_ref, out_ref, res_ref = rest
    else:
        xs_ref = None
        out_ref, res_ref = rest
    cnt = c0_ref[:, :1] + c1_ref[:, :1]
    agg = (ns0_ref[...] + ns1_ref[...]) / jnp.maximum(cnt, 1.0)
    h = jnp.dot(x_ref[...], ax_ref[...], preferred_element_type=jnp.float32)
    h = h + jnp.dot(agg, aa_ref[...], preferred_element_type=jnp.float32)
    h = _elu(h + tb_ref[:, H:])
    h = _elu(jnp.dot(h, ah_ref[...],
                     preferred_element_type=jnp.float32) + bh_ref[...])
    h = _elu(jnp.dot(h, ao_ref[...],
                     preferred_element_type=jnp.float32) + bo_ref[...])
    out_ref[...] = h
    if xs_ref is not None:
        res_ref[...] = h + xs_ref[...]
    else:
        res_ref[...] = h


def _node_mlp(x, ns0, ns1, c0, c1, Tb, Ax, Aa, Ah, bh, Ao, bo, xs=None):
    N, F = x.shape
    OUT = Ao.shape[1]
    AD = Aa.shape[0]
    BN = 2000
    ins = [
        pl.BlockSpec((BN, F), lambda i: (i, 0)),
        pl.BlockSpec((BN, AD), lambda i: (i, 0)),
        pl.BlockSpec((BN, AD), lambda i: (i, 0)),
        pl.BlockSpec((BN, 16), lambda i: (i, 0)),
        pl.BlockSpec((BN, 16), lambda i: (i, 0)),
        pl.BlockSpec((BN, 2 * H), lambda i: (i, 0)),
        pl.BlockSpec((F, H), lambda i: (0, 0)),
        pl.BlockSpec((AD, H), lambda i: (0, 0)),
        pl.BlockSpec((H, H), lambda i: (0, 0)),
        pl.BlockSpec((1, H), lambda i: (0, 0)),
        pl.BlockSpec((H, OUT), lambda i: (0, 0)),
        pl.BlockSpec((1, OUT), lambda i: (0, 0)),
    ]
    args = [x, ns0, ns1, c0, c1, Tb, Ax, Aa, Ah, bh, Ao, bo]
    if xs is not None:
        ins.append(pl.BlockSpec((BN, OUT), lambda i: (i, 0)))
        args.append(xs)
    return pl.pallas_call(
        _node_body,
        grid=(N // BN,),
        in_specs=ins,
        out_specs=[pl.BlockSpec((BN, OUT), lambda i: (i, 0))] * 2,
        out_shape=[jax.ShapeDtypeStruct((N, OUT), jnp.float32)] * 2,
        interpret=INTERPRET,
    )(*args)


def _global_body(u_ref, gx0_ref, gx1_ref, ge0_ref, ge1_ref, cb0_ref, cb1_ref,
                 cbr0_ref, cbr1_ref,
                 wu_ref, wx_ref, we_ref, b1_ref, wh_ref, bh_ref, wo_ref,
                 bo_ref, *rest):
    if len(rest) == 2:
        us_ref, out_ref = rest
    else:
        us_ref = None
        (out_ref,) = rest
    cb = cb0_ref[:, :1] + cb1_ref[:, :1]
    cbr = cbr0_ref[:, :1] + cbr1_ref[:, :1]
    mx = (gx0_ref[...] + gx1_ref[...]) / jnp.maximum(cb, 1.0)
    me = (ge0_ref[...] + ge1_ref[...]) / jnp.maximum(cbr, 1.0)
    h = jnp.dot(u_ref[...], wu_ref[...], preferred_element_type=jnp.float32)
    h = h + jnp.dot(mx, wx_ref[...], preferred_element_type=jnp.float32)
    h = h + jnp.dot(me, we_ref[...], preferred_element_type=jnp.float32)
    h = _elu(h + b1_ref[...])
    h = _elu(jnp.dot(h, wh_ref[...],
                     preferred_element_type=jnp.float32) + bh_ref[...])
    h = _elu(jnp.dot(h, wo_ref[...],
                     preferred_element_type=jnp.float32) + bo_ref[...])
    if us_ref is not None:
        h = h + us_ref[...]
    out_ref[...] = h


def _global_mlp(u, gx0, gx1, ge0, ge1, cb, cbr, Gu, Gx, Ge, b1, Wh, bh, Wo,
                bo, us=None):
    cb0, cb1 = cb
    cbr0, cbr1 = cbr
    G_, GF_ = u.shape
    XD = Gx.shape[0]
    ED = Ge.shape[0]
    OUT = Wo.shape[1]
    ins = [
        pl.BlockSpec((G_, GF_), lambda i: (0, 0)),
        pl.BlockSpec((G_, XD), lambda i: (0, 0)),
        pl.BlockSpec((G_, XD), lambda i: (0, 0)),
        pl.BlockSpec((G_, ED), lambda i: (0, 0)),
        pl.BlockSpec((G_, ED), lambda i: (0, 0)),
        pl.BlockSpec((G_, 16), lambda i: (0, 0)),
        pl.BlockSpec((G_, 16), lambda i: (0, 0)),
        pl.BlockSpec((G_, 16), lambda i: (0, 0)),
        pl.BlockSpec((G_, 16), lambda i: (0, 0)),
        pl.BlockSpec((GF_, H), lambda i: (0, 0)),
        pl.BlockSpec((XD, H), lambda i: (0, 0)),
        pl.BlockSpec((ED, H), lambda i: (0, 0)),
        pl.BlockSpec((1, H), lambda i: (0, 0)),
        pl.BlockSpec((H, H), lambda i: (0, 0)),
        pl.BlockSpec((1, H), lambda i: (0, 0)),
        pl.BlockSpec((H, OUT), lambda i: (0, 0)),
        pl.BlockSpec((1, OUT), lambda i: (0, 0)),
    ]
    args = [u, gx0, gx1, ge0, ge1, cb0, cb1, cbr0, cbr1,
            Gu, Gx, Ge, b1, Wh, bh, Wo, bo]
    if us is not None:
        ins.append(pl.BlockSpec((G_, OUT), lambda i: (0, 0)))
        args.append(us)
    return pl.pallas_call(
        _global_body,
        grid=(1,),
        in_specs=ins,
        out_specs=pl.BlockSpec((G_, OUT), lambda i: (0, 0)),
        out_shape=jax.ShapeDtypeStruct((G_, OUT), jnp.float32),
        interpret=INTERPRET,
    )(*args)


def _sc_segment_sum(data, idx2ds, nsegs, B):
    E, D = data.shape
    NB = E // B
    assert NB * B == E
    base, rem = NB // NW, NB % NW
    ntgt = len(nsegs)
    GP = 8
    gp = min(GP, max(base, 1))
    nr, left = base // gp, base % gp
    npads = [-(-n // 128) * 128 for n in nsegs]
    mesh = plsc.VectorSubcoreMesh(core_axis_name="c", subcore_axis_name="s")
    scratch = [pltpu.VMEM((max(base, 1), B), jnp.int32) for _ in range(ntgt)]
    scratch += [pltpu.VMEM((B,), jnp.int32) for _ in range(ntgt)]
    scratch += [pltpu.VMEM((gp * B, D), jnp.float32),
                pltpu.SemaphoreType.DMA]
    scratch += [pltpu.VMEM_SHARED((n, D), jnp.float32) for n in npads]

    @functools.partial(
        pl.kernel,
        out_type=[jax.ShapeDtypeStruct((NC, n, D), jnp.float32)
                  for n in npads],
        mesh=mesh,
        scratch_types=scratch,
        compiler_params=pltpu.CompilerParams(use_tc_tiling_on_sc=False),
    )
    def body(*refs):
        data_hbm = refs[0]
        idx_hbms = refs[1:1 + ntgt]
        zeros_hbm = refs[1 + ntgt]
        out_hbms = refs[2 + ntgt:2 + 2 * ntgt]
        r = 2 + 2 * ntgt
        idxall_vs = refs[r:r + ntgt]
        idxe_vs = refs[r + ntgt:r + 2 * ntgt]
        rows_v = refs[r + 2 * ntgt]
        sem = refs[r + 2 * ntgt + 1]
        accs = refs[r + 2 * ntgt + 2:]
        cid = lax.axis_index("c")
        sid = lax.axis_index("s")
        wid = sid * NC + cid
        for acc, n in zip(accs, npads):
            rpt = n // NS
            pltpu.sync_copy(zeros_hbm.at[pl.ds(0, rpt)],
                            acc.at[pl.ds(sid * rpt, rpt)])
        plsc.subcore_barrier()
        start = wid * base + jnp.minimum(wid, rem)
        if base:
            for idx_hbm, idxall_v in zip(idx_hbms, idxall_vs):
                pltpu.sync_copy(idx_hbm.at[pl.ds(start, base)], idxall_v)

        def group(i, carry):
            pltpu.sync_copy(data_hbm.at[pl.ds((start + i * gp) * B, gp * B)],
                            rows_v)
            descs = []
            for j in range(gp):
                lb = i * gp + j
                for t in range(ntgt):
                    descs.append(pltpu.async_copy(
                        rows_v.at[pl.ds(j * B, B)],
                        accs[t].at[idxall_vs[t].at[lb]], sem, add=True))
            for d in descs:
                d.wait()
            return carry

        if nr:
            lax.fori_loop(0, nr, group, 0)
        for j in range(left):
            lb = nr * gp + j
            pltpu.sync_copy(data_hbm.at[pl.ds((start + lb) * B, B)],
                            rows_v.at[pl.ds(0, B)])
            for t in range(ntgt):
                pltpu.sync_copy(rows_v.at[pl.ds(0, B)],
                                accs[t].at[idxall_vs[t].at[lb]], add=True)
        if rem:
            @pl.when(wid < rem)
            def _():
                b = start + base
                pltpu.sync_copy(data_hbm.at[pl.ds(b * B, B)],
                                rows_v.at[pl.ds(0, B)])
                for t in range(ntgt):
                    pltpu.sync_copy(idx_hbms[t].at[b], idxe_vs[t])
                    pltpu.sync_copy(rows_v.at[pl.ds(0, B)],
                                    accs[t].at[idxe_vs[t]], add=True)
        plsc.subcore_barrier()
        for acc, out_hbm, n in zip(accs, out_hbms, npads):
            rpt = n // NS
            pltpu.sync_copy(acc.at[pl.ds(sid * rpt, rpt)],
                            out_hbm.at[cid, pl.ds(sid * rpt, rpt)])

    zeros = jnp.zeros((max(npads) // NS, D), jnp.float32)
    outs = body(data, *idx2ds, zeros)
    if not isinstance(outs, (list, tuple)):
        outs = [outs]
    return list(outs)


def _sc_gather(tabs, idx2ds, D, GP=4):
    NB, B = idx2ds[0].shape
    E = NB * B
    base, rem = NB // NW, NB % NW
    nt = len(tabs)
    gp = min(GP, max(base, 1))
    nr, left = base // gp, base % gp
    mesh = plsc.VectorSubcoreMesh(core_axis_name="c", subcore_axis_name="s")
    scratch = [pltpu.VMEM((max(base, 1), B), jnp.int32) for _ in range(nt)]
    scratch += [pltpu.VMEM((B,), jnp.int32) for _ in range(nt)]
    scratch += [pltpu.VMEM((gp * B, D), jnp.float32) for _ in range(nt)]
    scratch += [pltpu.SemaphoreType.DMA for _ in range(nt)]

    @functools.partial(
        pl.kernel,
        out_type=[jax.ShapeDtypeStruct((E, D), jnp.float32)
                  for _ in range(nt)],
        mesh=mesh,
        scratch_types=scratch,
        compiler_params=pltpu.CompilerParams(use_tc_tiling_on_sc=False),
    )
    def body(*refs):
        tab_hbms = refs[:nt]
        idx_hbms = refs[nt:2 * nt]
        out_hbms = refs[2 * nt:3 * nt]
        r = 3 * nt
        idxall_vs = refs[r:r + nt]
        idxe_vs = refs[r + nt:r + 2 * nt]
        buf_vs = refs[r + 2 * nt:r + 3 * nt]
        sems = refs[r + 3 * nt:]
        cid = lax.axis_index("c")
        sid = lax.axis_index("s")
        wid = sid * NC + cid
        start = wid * base + jnp.minimum(wid, rem)
        if base:
            for idx_hbm, idxall_v in zip(idx_hbms, idxall_vs):
                pltpu.sync_copy(idx_hbm.at[pl.ds(start, base)], idxall_v)

        def group(i, carry):
            descs = []
            for j in range(gp):
                lb = i * gp + j
                for t in range(nt):
                    descs.append(pltpu.async_copy(
                        tab_hbms[t].at[idxall_vs[t].at[lb]],
                        buf_vs[t].at[pl.ds(j * B, B)], sems[t]))
            for d in descs:
                d.wait()
            for t in range(nt):
                pltpu.sync_copy(
                    buf_vs[t],
                    out_hbms[t].at[pl.ds((start + i * gp) * B, gp * B)])
            return carry

        if nr:
            lax.fori_loop(0, nr, group, 0)
        for j in range(left):
            lb = nr * gp + j
            for t in range(nt):
                pltpu.async_copy(tab_hbms[t].at[idxall_vs[t].at[lb]],
                                 buf_vs[t].at[pl.ds(0, B)], sems[t]).wait()
                pltpu.sync_copy(buf_vs[t].at[pl.ds(0, B)],
                                out_hbms[t].at[pl.ds((start + lb) * B, B)])
        if rem:
            @pl.when(wid < rem)
            def _():
                b = start + base
                for t in range(nt):
                    pltpu.sync_copy(idx_hbms[t].at[b], idxe_vs[t])
                    pltpu.async_copy(tab_hbms[t].at[idxe_vs[t]],
                                     buf_vs[t].at[pl.ds(0, B)],
                                     sems[t]).wait()
                    pltpu.sync_copy(buf_vs[t].at[pl.ds(0, B)],
                                    out_hbms[t].at[pl.ds(b * B, B)])

    outs = body(*tabs, *idx2ds)
    if not isinstance(outs, (list, tuple)):
        outs = [outs]
    return list(outs)


def _scatter_edges(e_out, col2d, birow2d, N, NG):
    ns, ge = _sc_segment_sum(e_out, [col2d, birow2d], [N, NG], 128)
    return ns[0], ns[1], ge[0], ge[1]


def _scatter_nodes(x_out, batch2d, NG):
    (gx,) = _sc_segment_sum(x_out, [batch2d], [NG], 80)
    return gx[0], gx[1]


def _sc_precompute(row2d, col2d, batch, batch2d, N, NG):
    NB, B = row2d.shape
    NB2, B2 = batch2d.shape
    NPAD = -(-N // 128) * 128
    base, rem = NB // NW, NB % NW
    base2, rem2 = NB2 // NW, NB2 % NW
    gp = min(4, max(base, 1))
    nr, left = base // gp, base % gp
    CW = 16
    mesh = plsc.VectorSubcoreMesh(core_axis_name="c", subcore_axis_name="s")

    @functools.partial(
        pl.kernel,
        out_type=[
            jax.ShapeDtypeStruct((NB, B), jnp.int32),
            jax.ShapeDtypeStruct((NC, NPAD, CW), jnp.float32),
            jax.ShapeDtypeStruct((NC, 128, CW), jnp.float32),
            jax.ShapeDtypeStruct((NC, 128, CW), jnp.float32),
        ],
        mesh=mesh,
        scratch_types=[
            pltpu.VMEM((max(base, 1), B), jnp.int32),
            pltpu.VMEM((max(base, 1), B), jnp.int32),
            pltpu.VMEM((gp, B), jnp.int32),
            pltpu.VMEM((B,), jnp.int32),
            pltpu.VMEM((B,), jnp.int32),
            pltpu.VMEM((B2,), jnp.int32),
            pltpu.VMEM((B, CW), jnp.float32),
            pltpu.SemaphoreType.DMA,
            pltpu.SemaphoreType.DMA,
            pltpu.VMEM_SHARED((NPAD, CW), jnp.float32),
            pltpu.VMEM_SHARED((128, CW), jnp.float32),
            pltpu.VMEM_SHARED((128, CW), jnp.float32),
        ],
        compiler_params=pltpu.CompilerParams(use_tc_tiling_on_sc=False),
    )
    def body(row_hbm, col_hbm, batch_hbm, zeros_hbm,
             birow_hbm, ccol_hbm, cb_hbm, cbr_hbm,
             idxar_v, idxac_v, bir_v, idxr_v, idxc_v, idxb_v, ones_v,
             semg, sems, ccol_s, cb_s, cbr_s):
        cid = lax.axis_index("c")
        sid = lax.axis_index("s")
        wid = sid * NC + cid

        def fill(r, c):
            ones_v[r, pl.ds(0, CW)] = jnp.full((CW,), 1.0, jnp.float32)
            return c

        lax.fori_loop(0, B, fill, 0)
        rpt = NPAD // NS
        pltpu.sync_copy(zeros_hbm.at[pl.ds(0, rpt)],
                        ccol_s.at[pl.ds(sid * rpt, rpt)])
        pltpu.sync_copy(zeros_hbm.at[pl.ds(0, 8)],
                        cb_s.at[pl.ds(sid * 8, 8)])
        pltpu.sync_copy(zeros_hbm.at[pl.ds(0, 8)],
                        cbr_s.at[pl.ds(sid * 8, 8)])
        plsc.subcore_barrier()

        start = wid * base + jnp.minimum(wid, rem)
        if base:
            pltpu.sync_copy(row_hbm.at[pl.ds(start, base)], idxar_v)
            pltpu.sync_copy(col_hbm.at[pl.ds(start, base)], idxac_v)

        def group(i, carry):
            descs = [pltpu.async_copy(batch_hbm.at[idxar_v.at[i * gp + j]],
                                      bir_v.at[j], semg)
                     for j in range(gp)]
            for d in descs:
                d.wait()
            pltpu.sync_copy(bir_v, birow_hbm.at[pl.ds(start + i * gp, gp)])
            descs = []
            for j in range(gp):
                lb = i * gp + j
                descs.append(pltpu.async_copy(
                    ones_v, ccol_s.at[idxac_v.at[lb]], sems, add=True))
                descs.append(pltpu.async_copy(
                    ones_v, cbr_s.at[bir_v.at[j]], sems, add=True))
            for d in descs:
                d.wait()
            return carry

        if nr:
            lax.fori_loop(0, nr, group, 0)
        for j in range(left):
            lb = nr * gp + j
            pltpu.async_copy(batch_hbm.at[idxar_v.at[lb]], bir_v.at[0],
                             semg).wait()
            pltpu.sync_copy(bir_v.at[pl.ds(0, 1)],
                            birow_hbm.at[pl.ds(start + lb, 1)])
            pltpu.sync_copy(ones_v, ccol_s.at[idxac_v.at[lb]], add=True)
            pltpu.sync_copy(ones_v, cbr_s.at[bir_v.at[0]], add=True)
        if rem:
            @pl.when(wid < rem)
            def _():
                b = start + base
                pltpu.sync_copy(row_hbm.at[b], idxr_v)
                pltpu.sync_copy(col_hbm.at[b], idxc_v)
                pltpu.async_copy(batch_hbm.at[idxr_v], bir_v.at[0],
                                 semg).wait()
                pltpu.sync_copy(bir_v.at[pl.ds(0, 1)],
                                birow_hbm.at[pl.ds(b, 1)])
                pltpu.sync_copy(ones_v, ccol_s.at[idxc_v], add=True)
                pltpu.sync_copy(ones_v, cbr_s.at[bir_v.at[0]], add=True)

        start2 = wid * base2 + jnp.minimum(wid, rem2)
        nb2 = base2 + jnp.where(wid < rem2, 1, 0)

        def blk2(i, carry):
            b = start2 + i
            pltpu.sync_copy(batch_hbm.at[pl.ds(b * B2, B2)], idxb_v)
            pltpu.sync_copy(ones_v.at[pl.ds(0, B2)], cb_s.at[idxb_v],
                            add=True)
            return carry

        lax.fori_loop(0, nb2, blk2, 0)
        plsc.subcore_barrier()
        pltpu.sync_copy(ccol_s.at[pl.ds(sid * rpt, rpt)],
                        ccol_hbm.at[cid, pl.ds(sid * rpt, rpt)])
        pltpu.sync_copy(cb_s.at[pl.ds(sid * 8, 8)],
                        cb_hbm.at[cid, pl.ds(sid * 8, 8)])
        pltpu.sync_copy(cbr_s.at[pl.ds(sid * 8, 8)],
                        cbr_hbm.at[cid, pl.ds(sid * 8, 8)])

    zeros = jnp.zeros((NPAD // NS, CW), jnp.float32)
    return body(row2d, col2d, batch, zeros)


def _pad_cols(w, width):
    return jnp.pad(w, ((0, 0), (0, width - w.shape[1])))


def _split_in(W, dims):
    out, o = [], 0
    for d in dims:
        out.append(W[o:o + d])
        o += d
    return out


def _meta_layer(p, x, ea, u, row2d, col2d, birow2d, batch2d,
                cnt_col, cnt_b, cnt_br,
                e_pad, x_pad, g_pad, ea_prev=None, xs=None, us=None):
    N, nf = x.shape
    E, ef = ea.shape
    NG, gf = u.shape

    e_dim_real = p["edge"]["lin_out"]["W"].shape[1]
    We, Wr, Wc, Wu = _split_in(p["edge"]["lin_in"]["W"], (ef, nf, nf, gf))
    Ax, Aa, Au = _split_in(p["node"]["lin_in"]["W"], (nf, e_dim_real, gf))
    Aa = jnp.pad(Aa, ((0, e_pad - e_dim_real), (0, 0)))
    Gu = p["global"]["lin_in"]["W"][:gf]

    Wut = jnp.concatenate([Wu, Au], axis=1)
    but = jnp.concatenate([p["edge"]["lin_in"]["b"],
                           p["node"]["lin_in"]["b"]])[None, :]
    T = _tmat(u, Wut, but)
    Tb = _sc_gather([T], [batch2d], 2 * H)[0]
    XRU, XC = _proj(x, jnp.concatenate([Wr, Wc], axis=1), Tb)
    G0, G1 = _sc_gather([XRU, XC], [row2d, col2d], H)

    eW_hid = p["edge"]["lin_hid"]
    eW_out = p["edge"]["lin_out"]
    e_out = _edge_mlp(
        ea, G0, G1, We, eW_hid["W"], eW_hid["b"][None, :],
        _pad_cols(eW_out["W"], e_pad), _pad_cols(eW_out["b"][None, :], e_pad),
        ea_prev=ea_prev)

    ns0, ns1, ge0, ge1 = _scatter_edges(e_out, col2d, birow2d, N, NG)

    nW_hid = p["node"]["lin_hid"]
    nW_out = p["node"]["lin_out"]
    x_fresh, x_res = _node_mlp(
        x, ns0, ns1, cnt_col[0], cnt_col[1], Tb, Ax, Aa,
        nW_hid["W"], nW_hid["b"][None, :],
        _pad_cols(nW_out["W"], x_pad), _pad_cols(nW_out["b"][None, :], x_pad),
        xs=xs)

    gx0, gx1 = _scatter_nodes(x_fresh, batch2d, NG)

    Wg = p["global"]["lin_in"]["W"]
    e_dim_real = p["edge"]["lin_out"]["W"].shape[1]
    x_dim_real = Wg.shape[0] - gf - e_dim_real
    Gx_r = Wg[gf:gf + x_dim_real]
    Ge_r = Wg[gf + x_dim_real:]
    Gx_pad = jnp.pad(Gx_r, ((0, x_pad - Gx_r.shape[0]), (0, 0)))
    Ge_pad = jnp.pad(Ge_r, ((0, e_pad - Ge_r.shape[0]), (0, 0)))

    gW_hid = p["global"]["lin_hid"]
    gW_out = p["global"]["lin_out"]
    u_out = _global_mlp(
        u, gx0, gx1, ge0, ge1, cnt_b, cnt_br, Gu, Gx_pad, Ge_pad,
        p["global"]["lin_in"]["b"][None, :], gW_hid["W"], gW_hid["b"][None, :],
        _pad_cols(gW_out["W"], g_pad), _pad_cols(gW_out["b"][None, :], g_pad),
        us=us)

    return x_fresh, x_res, e_out, u_out


def kernel(x, u, edge_index, edge_attr, batch, params):
    N, _ = x.shape
    NG, _ = u.shape
    E = edge_attr.shape[0]
    row, col = edge_index[0], edge_index[1]

    row2d = row.reshape(E // 128, 128)
    col2d = col.reshape(E // 128, 128)
    batch2d = batch.reshape(N // 80, 80)
    birow2d, ccol_p, cb_p, cbr_p = _sc_precompute(
        row2d, col2d, batch, batch2d, N, NG)
    cnt_col = (ccol_p[0], ccol_p[1])
    cnt_b = (cb_p[0], cb_p[1])
    cnt_br = (cbr_p[0], cbr_p[1])

    pf, pm, plast = params["first"], params["mid"][0], params["last"]

    x1, _, e1, u1 = _meta_layer(
        pf, x, edge_attr, u, row2d, col2d, birow2d, batch2d,
        cnt_col, cnt_b, cnt_br, e_pad=H, x_pad=H, g_pad=H)

    x2f, x2, e2, u2 = _meta_layer(
        pm, x1, e1, u1, row2d, col2d, birow2d, batch2d,
        cnt_col, cnt_b, cnt_br, e_pad=H, x_pad=H, g_pad=H, xs=x1, us=u1)

    x3, _, e3, u3 = _meta_layer(
        plast, x2, e2, u2, row2d, col2d, birow2d, batch2d,
        cnt_col, cnt_b, cnt_br, e_pad=16, x_pad=16, g_pad=8, ea_prev=e1)

    return (x3[:, :7], e3[:, :6], u3[:, :1])


# speedup vs baseline: 1.0711x; 1.0711x over previous
import functools

import jax
import jax.numpy as jnp
from jax import lax
from jax.experimental import pallas as pl
from jax.experimental.pallas import tpu as pltpu
from jax.experimental.pallas import tpu_sc as plsc

INTERPRET = False

H = 64
NC, NS = 2, 16
NW = NC * NS


def _elu(h):
    return jnp.where(h > 0.0, h, jnp.exp(h) - 1.0)


def _onehot_body(b_ref, oh_ref):
    ids = jax.lax.broadcasted_iota(jnp.int32, oh_ref.shape, 1)
    oh_ref[...] = jnp.where(b_ref[...] == ids, 1.0, 0.0).astype(jnp.float32)


def _onehot(batch_col, NG):
    N = batch_col.shape[0]
    BN = 2000
    return pl.pallas_call(
        _onehot_body,
        grid=(N // BN,),
        in_specs=[pl.BlockSpec((BN, 1), lambda i: (i, 0))],
        out_specs=pl.BlockSpec((BN, NG), lambda i: (i, 0)),
        out_shape=jax.ShapeDtypeStruct((N, NG), jnp.float32),
        interpret=INTERPRET,
    )(batch_col)


def _proj_body(x_ref, w_ref, oh_ref, t_ref, xr_ref, xc_ref):
    y = jnp.dot(x_ref[...], w_ref[...], preferred_element_type=jnp.float32)
    tb = jnp.dot(oh_ref[...], t_ref[:, :H], preferred_element_type=jnp.float32)
    xr_ref[...] = y[:, :H] + tb
    xc_ref[...] = y[:, H:]


def _proj(x, Wrc, OH, T):
    N, F = x.shape
    BN = 2000
    return pl.pallas_call(
        _proj_body,
        grid=(N // BN,),
        in_specs=[
            pl.BlockSpec((BN, F), lambda i: (i, 0)),
            pl.BlockSpec((F, 2 * H), lambda i: (0, 0)),
            pl.BlockSpec((BN, 64), lambda i: (i, 0)),
            pl.BlockSpec((64, 2 * H), lambda i: (0, 0)),
        ],
        out_specs=[
            pl.BlockSpec((BN, H), lambda i: (i, 0)),
            pl.BlockSpec((BN, H), lambda i: (i, 0)),
        ],
        out_shape=[
            jax.ShapeDtypeStruct((N, H), jnp.float32),
            jax.ShapeDtypeStruct((N, H), jnp.float32),
        ],
        interpret=INTERPRET,
    )(x, Wrc, OH, T)


def _tmat_body(u_ref, w_ref, b_ref, t_ref):
    t_ref[...] = jnp.dot(u_ref[...], w_ref[...],
                         preferred_element_type=jnp.float32) + b_ref[...]


def _tmat(u, Wut, but):
    G_, GF_ = u.shape
    return pl.pallas_call(
        _tmat_body,
        grid=(1,),
        in_specs=[
            pl.BlockSpec((G_, GF_), lambda i: (0, 0)),
            pl.BlockSpec((GF_, 2 * H), lambda i: (0, 0)),
            pl.BlockSpec((1, 2 * H), lambda i: (0, 0)),
        ],
        out_specs=pl.BlockSpec((G_, 2 * H), lambda i: (0, 0)),
        out_shape=jax.ShapeDtypeStruct((G_, 2 * H), jnp.float32),
        interpret=INTERPRET,
    )(u, Wut, but)


def _edge_body(ea_ref, g0_ref, g1_ref, we_ref, wh_ref, bh_ref, wo_ref,
               bo_ref, out_ref):
    h = _elu(jnp.dot(ea_ref[...], we_ref[...],
                     preferred_element_type=jnp.float32)
             + g0_ref[...] + g1_ref[...])
    h = _elu(jnp.dot(h, wh_ref[...],
                     preferred_element_type=jnp.float32) + bh_ref[...])
    h = _elu(jnp.dot(h, wo_ref[...],
                     preferred_element_type=jnp.float32) + bo_ref[...])
    out_ref[...] = h


def _edge_body_prev(ea_ref, ep_ref, g0_ref, g1_ref, we_ref, wh_ref, bh_ref,
                    wo_ref, bo_ref, out_ref):
    h = _elu(jnp.dot(ea_ref[...] + ep_ref[...], we_ref[...],
                     preferred_element_type=jnp.float32)
             + g0_ref[...] + g1_ref[...])
    h = _elu(jnp.dot(h, wh_ref[...],
                     preferred_element_type=jnp.float32) + bh_ref[...])
    h = _elu(jnp.dot(h, wo_ref[...],
                     preferred_element_type=jnp.float32) + bo_ref[...])
    out_ref[...] = h


def _edge_mlp(ea, G0, G1, We, Wh, bh, Wo, bo, ea_prev=None):
    E, EFd = ea.shape
    OUT = Wo.shape[1]
    BE = 2000
    wspecs = [
        pl.BlockSpec((EFd, H), lambda i: (0, 0)),
        pl.BlockSpec((H, H), lambda i: (0, 0)),
        pl.BlockSpec((1, H), lambda i: (0, 0)),
        pl.BlockSpec((H, OUT), lambda i: (0, 0)),
        pl.BlockSpec((1, OUT), lambda i: (0, 0)),
    ]
    dspec = pl.BlockSpec((BE, EFd), lambda i: (i, 0))
    gspec = pl.BlockSpec((BE, H), lambda i: (i, 0))
    if ea_prev is None:
        body, args, ins = _edge_body, (ea, G0, G1), [dspec, gspec, gspec] + wspecs
    else:
        body, args, ins = (_edge_body_prev, (ea, ea_prev, G0, G1),
                           [dspec, dspec, gspec, gspec] + wspecs)
    return pl.pallas_call(
        body,
        grid=(E // BE,),
        in_specs=ins,
        out_specs=pl.BlockSpec((BE, OUT), lambda i: (i, 0)),
        out_shape=jax.ShapeDtypeStruct((E, OUT), jnp.float32),
        interpret=INTERPRET,
    )(*args, We, Wh, bh, Wo, bo)


def _node_body(x_ref, ns0_ref, ns1_ref, c0_ref, c1_ref, oh_ref, t_ref,
               ax_ref, aa_ref, ah_ref, bh_ref, ao_ref, bo_ref, *rest):
    if len(rest) == 4:
        xs_ref, out_ref, res_ref, gx_ref = rest
    else:
        xs_ref = None
        out_ref, res_ref, gx_ref = rest
    cnt = c0_ref[:, :1] + c1_ref[:, :1]
    agg = (ns0_ref[...] + ns1_ref[...]) / jnp.maximum(cnt, 1.0)
    h = jnp.dot(x_ref[...], ax_ref[...], preferred_element_type=jnp.float32)
    h = h + jnp.dot(agg, aa_ref[...], preferred_element_type=jnp.float32)
    un = jnp.dot(oh_ref[...], t_ref[:, H:], preferred_element_type=jnp.float32)
    h = _elu(h + un)
    h = _elu(jnp.dot(h, ah_ref[...],
                     preferred_element_type=jnp.float32) + bh_ref[...])
    h = _elu(jnp.dot(h, ao_ref[...],
                     preferred_element_type=jnp.float32) + bo_ref[...])
    out_ref[...] = h
    if xs_ref is not None:
        res_ref[...] = h + xs_ref[...]
    else:
        res_ref[...] = h
    part = jax.lax.dot_general(oh_ref[...], h, (((0,), (0,)), ((), ())),
                               preferred_element_type=jnp.float32)

    @pl.when(pl.program_id(0) == 0)
    def _():
        gx_ref[...] = jnp.zeros_like(gx_ref)

    gx_ref[...] += part


def _node_mlp(x, ns0, ns1, c0, c1, OH, T, Ax, Aa, Ah, bh, Ao, bo, xs=None):
    N, F = x.shape
    OUT = Ao.shape[1]
    AD = Aa.shape[0]
    BN = 2000
    ins = [
        pl.BlockSpec((BN, F), lambda i: (i, 0)),
        pl.BlockSpec((BN, AD), lambda i: (i, 0)),
        pl.BlockSpec((BN, AD), lambda i: (i, 0)),
        pl.BlockSpec((BN, 16), lambda i: (i, 0)),
        pl.BlockSpec((BN, 16), lambda i: (i, 0)),
        pl.BlockSpec((BN, 64), lambda i: (i, 0)),
        pl.BlockSpec((64, 2 * H), lambda i: (0, 0)),
        pl.BlockSpec((F, H), lambda i: (0, 0)),
        pl.BlockSpec((AD, H), lambda i: (0, 0)),
        pl.BlockSpec((H, H), lambda i: (0, 0)),
        pl.BlockSpec((1, H), lambda i: (0, 0)),
        pl.BlockSpec((H, OUT), lambda i: (0, 0)),
        pl.BlockSpec((1, OUT), lambda i: (0, 0)),
    ]
    args = [x, ns0, ns1, c0, c1, OH, T, Ax, Aa, Ah, bh, Ao, bo]
    if xs is not None:
        ins.append(pl.BlockSpec((BN, OUT), lambda i: (i, 0)))
        args.append(xs)
    return pl.pallas_call(
        _node_body,
        grid=(N // BN,),
        in_specs=ins,
        out_specs=[pl.BlockSpec((BN, OUT), lambda i: (i, 0))] * 2
        + [pl.BlockSpec((64, OUT), lambda i: (0, 0))],
        out_shape=[jax.ShapeDtypeStruct((N, OUT), jnp.float32)] * 2
        + [jax.ShapeDtypeStruct((64, OUT), jnp.float32)],
        interpret=INTERPRET,
    )(*args)


def _global_body(u_ref, gx_ref, ge0_ref, ge1_ref, cb0_ref, cb1_ref,
                 cbr0_ref, cbr1_ref,
                 wu_ref, wx_ref, we_ref, b1_ref, wh_ref, bh_ref, wo_ref,
                 bo_ref, *rest):
    if len(rest) == 2:
        us_ref, out_ref = rest
    else:
        us_ref = None
        (out_ref,) = rest
    cb = cb0_ref[:, :1] + cb1_ref[:, :1]
    cbr = cbr0_ref[:, :1] + cbr1_ref[:, :1]
    mx = gx_ref[...] / jnp.maximum(cb, 1.0)
    me = (ge0_ref[...] + ge1_ref[...]) / jnp.maximum(cbr, 1.0)
    h = jnp.dot(u_ref[...], wu_ref[...], preferred_element_type=jnp.float32)
    h = h + jnp.dot(mx, wx_ref[...], preferred_element_type=jnp.float32)
    h = h + jnp.dot(me, we_ref[...], preferred_element_type=jnp.float32)
    h = _elu(h + b1_ref[...])
    h = _elu(jnp.dot(h, wh_ref[...],
                     preferred_element_type=jnp.float32) + bh_ref[...])
    h = _elu(jnp.dot(h, wo_ref[...],
                     preferred_element_type=jnp.float32) + bo_ref[...])
    if us_ref is not None:
        h = h + us_ref[...]
    out_ref[...] = h


def _global_mlp(u, gx, ge0, ge1, cb, cbr, Gu, Gx, Ge, b1, Wh, bh, Wo,
                bo, us=None):
    cb0, cb1 = cb
    cbr0, cbr1 = cbr
    G_, GF_ = u.shape
    XD = Gx.shape[0]
    ED = Ge.shape[0]
    OUT = Wo.shape[1]
    ins = [
        pl.BlockSpec((G_, GF_), lambda i: (0, 0)),
        pl.BlockSpec((G_, XD), lambda i: (0, 0)),
        pl.BlockSpec((G_, ED), lambda i: (0, 0)),
        pl.BlockSpec((G_, ED), lambda i: (0, 0)),
        pl.BlockSpec((G_, 16), lambda i: (0, 0)),
        pl.BlockSpec((G_, 16), lambda i: (0, 0)),
        pl.BlockSpec((G_, 16), lambda i: (0, 0)),
        pl.BlockSpec((G_, 16), lambda i: (0, 0)),
        pl.BlockSpec((GF_, H), lambda i: (0, 0)),
        pl.BlockSpec((XD, H), lambda i: (0, 0)),
        pl.BlockSpec((ED, H), lambda i: (0, 0)),
        pl.BlockSpec((1, H), lambda i: (0, 0)),
        pl.BlockSpec((H, H), lambda i: (0, 0)),
        pl.BlockSpec((1, H), lambda i: (0, 0)),
        pl.BlockSpec((H, OUT), lambda i: (0, 0)),
        pl.BlockSpec((1, OUT), lambda i: (0, 0)),
    ]
    args = [u, gx, ge0, ge1, cb0, cb1, cbr0, cbr1,
            Gu, Gx, Ge, b1, Wh, bh, Wo, bo]
    if us is not None:
        ins.append(pl.BlockSpec((G_, OUT), lambda i: (0, 0)))
        args.append(us)
    return pl.pallas_call(
        _global_body,
        grid=(1,),
        in_specs=ins,
        out_specs=pl.BlockSpec((G_, OUT), lambda i: (0, 0)),
        out_shape=jax.ShapeDtypeStruct((G_, OUT), jnp.float32),
        interpret=INTERPRET,
    )(*args)


def _sc_segment_sum(data, idx2ds, nsegs, B):
    E, D = data.shape
    NB = E // B
    assert NB * B == E
    base, rem = NB // NW, NB % NW
    ntgt = len(nsegs)
    GP = 8
    gp = min(GP, max(base, 1))
    nr, left = base // gp, base % gp
    npads = [-(-n // 128) * 128 for n in nsegs]
    mesh = plsc.VectorSubcoreMesh(core_axis_name="c", subcore_axis_name="s")
    scratch = [pltpu.VMEM((max(base, 1), B), jnp.int32) for _ in range(ntgt)]
    scratch += [pltpu.VMEM((B,), jnp.int32) for _ in range(ntgt)]
    scratch += [pltpu.VMEM((gp * B, D), jnp.float32),
                pltpu.SemaphoreType.DMA]
    scratch += [pltpu.VMEM_SHARED((n, D), jnp.float32) for n in npads]

    @functools.partial(
        pl.kernel,
        out_type=[jax.ShapeDtypeStruct((NC, n, D), jnp.float32)
                  for n in npads],
        mesh=mesh,
        scratch_types=scratch,
        compiler_params=pltpu.CompilerParams(use_tc_tiling_on_sc=False),
    )
    def body(*refs):
        data_hbm = refs[0]
        idx_hbms = refs[1:1 + ntgt]
        zeros_hbm = refs[1 + ntgt]
        out_hbms = refs[2 + ntgt:2 + 2 * ntgt]
        r = 2 + 2 * ntgt
        idxall_vs = refs[r:r + ntgt]
        idxe_vs = refs[r + ntgt:r + 2 * ntgt]
        rows_v = refs[r + 2 * ntgt]
        sem = refs[r + 2 * ntgt + 1]
        accs = refs[r + 2 * ntgt + 2:]
        cid = lax.axis_index("c")
        sid = lax.axis_index("s")
        wid = sid * NC + cid
        for acc, n in zip(accs, npads):
            rpt = n // NS
            pltpu.sync_copy(zeros_hbm.at[pl.ds(0, rpt)],
                            acc.at[pl.ds(sid * rpt, rpt)])
        plsc.subcore_barrier()
        start = wid * base + jnp.minimum(wid, rem)
        if base:
            for idx_hbm, idxall_v in zip(idx_hbms, idxall_vs):
                pltpu.sync_copy(idx_hbm.at[pl.ds(start, base)], idxall_v)

        def group(i, carry):
            pltpu.sync_copy(data_hbm.at[pl.ds((start + i * gp) * B, gp * B)],
                            rows_v)
            descs = []
            for j in range(gp):
                lb = i * gp + j
                for t in range(ntgt):
                    descs.append(pltpu.async_copy(
                        rows_v.at[pl.ds(j * B, B)],
                        accs[t].at[idxall_vs[t].at[lb]], sem, add=True))
            for d in descs:
                d.wait()
            return carry

        if nr:
            lax.fori_loop(0, nr, group, 0)
        for j in range(left):
            lb = nr * gp + j
            pltpu.sync_copy(data_hbm.at[pl.ds((start + lb) * B, B)],
                            rows_v.at[pl.ds(0, B)])
            for t in range(ntgt):
                pltpu.sync_copy(rows_v.at[pl.ds(0, B)],
                                accs[t].at[idxall_vs[t].at[lb]], add=True)
        if rem:
            @pl.when(wid < rem)
            def _():
                b = start + base
                pltpu.sync_copy(data_hbm.at[pl.ds(b * B, B)],
                                rows_v.at[pl.ds(0, B)])
                for t in range(ntgt):
                    pltpu.sync_copy(idx_hbms[t].at[b], idxe_vs[t])
                    pltpu.sync_copy(rows_v.at[pl.ds(0, B)],
                                    accs[t].at[idxe_vs[t]], add=True)
        plsc.subcore_barrier()
        for acc, out_hbm, n in zip(accs, out_hbms, npads):
            rpt = n // NS
            pltpu.sync_copy(acc.at[pl.ds(sid * rpt, rpt)],
                            out_hbm.at[cid, pl.ds(sid * rpt, rpt)])

    zeros = jnp.zeros((max(npads) // NS, D), jnp.float32)
    outs = body(data, *idx2ds, zeros)
    if not isinstance(outs, (list, tuple)):
        outs = [outs]
    return list(outs)


def _sc_gather(tabs, idx2ds, D, GP=4):
    NB, B = idx2ds[0].shape
    E = NB * B
    base, rem = NB // NW, NB % NW
    nt = len(tabs)
    gp = min(GP, max(base, 1))
    nr, left = base // gp, base % gp
    mesh = plsc.VectorSubcoreMesh(core_axis_name="c", subcore_axis_name="s")
    scratch = [pltpu.VMEM((max(base, 1), B), jnp.int32) for _ in range(nt)]
    scratch += [pltpu.VMEM((B,), jnp.int32) for _ in range(nt)]
    scratch += [pltpu.VMEM((gp * B, D), jnp.float32) for _ in range(nt)]
    scratch += [pltpu.SemaphoreType.DMA for _ in range(nt)]

    @functools.partial(
        pl.kernel,
        out_type=[jax.ShapeDtypeStruct((E, D), jnp.float32)
                  for _ in range(nt)],
        mesh=mesh,
        scratch_types=scratch,
        compiler_params=pltpu.CompilerParams(use_tc_tiling_on_sc=False),
    )
    def body(*refs):
        tab_hbms = refs[:nt]
        idx_hbms = refs[nt:2 * nt]
        out_hbms = refs[2 * nt:3 * nt]
        r = 3 * nt
        idxall_vs = refs[r:r + nt]
        idxe_vs = refs[r + nt:r + 2 * nt]
        buf_vs = refs[r + 2 * nt:r + 3 * nt]
        sems = refs[r + 3 * nt:]
        cid = lax.axis_index("c")
        sid = lax.axis_index("s")
        wid = sid * NC + cid
        start = wid * base + jnp.minimum(wid, rem)
        if base:
            for idx_hbm, idxall_v in zip(idx_hbms, idxall_vs):
                pltpu.sync_copy(idx_hbm.at[pl.ds(start, base)], idxall_v)

        def group(i, carry):
            descs = []
            for j in range(gp):
                lb = i * gp + j
                for t in range(nt):
                    descs.append(pltpu.async_copy(
                        tab_hbms[t].at[idxall_vs[t].at[lb]],
                        buf_vs[t].at[pl.ds(j * B, B)], sems[t]))
            for d in descs:
                d.wait()
            for t in range(nt):
                pltpu.sync_copy(
                    buf_vs[t],
                    out_hbms[t].at[pl.ds((start + i * gp) * B, gp * B)])
            return carry

        if nr:
            lax.fori_loop(0, nr, group, 0)
        for j in range(left):
            lb = nr * gp + j
            for t in range(nt):
                pltpu.async_copy(tab_hbms[t].at[idxall_vs[t].at[lb]],
                                 buf_vs[t].at[pl.ds(0, B)], sems[t]).wait()
                pltpu.sync_copy(buf_vs[t].at[pl.ds(0, B)],
                                out_hbms[t].at[pl.ds((start + lb) * B, B)])
        if rem:
            @pl.when(wid < rem)
            def _():
                b = start + base
                for t in range(nt):
                    pltpu.sync_copy(idx_hbms[t].at[b], idxe_vs[t])
                    pltpu.async_copy(tab_hbms[t].at[idxe_vs[t]],
                                     buf_vs[t].at[pl.ds(0, B)],
                                     sems[t]).wait()
                    pltpu.sync_copy(buf_vs[t].at[pl.ds(0, B)],
                                    out_hbms[t].at[pl.ds(b * B, B)])

    outs = body(*tabs, *idx2ds)
    if not isinstance(outs, (list, tuple)):
        outs = [outs]
    return list(outs)


def _scatter_edges(e_out, col2d, birow2d, N, NG):
    ns, ge = _sc_segment_sum(e_out, [col2d, birow2d], [N, NG], 128)
    return ns[0], ns[1], ge[0], ge[1]


def _scatter_nodes(x_out, batch2d, NG):
    (gx,) = _sc_segment_sum(x_out, [batch2d], [NG], 80)
    return gx[0], gx[1]


def _sc_precompute(row2d, col2d, batch, batch2d, N, NG):
    NB, B = row2d.shape
    NB2, B2 = batch2d.shape
    NPAD = -(-N // 128) * 128
    base, rem = NB // NW, NB % NW
    base2, rem2 = NB2 // NW, NB2 % NW
    gp = min(4, max(base, 1))
    nr, left = base // gp, base % gp
    CW = 16
    mesh = plsc.VectorSubcoreMesh(core_axis_name="c", subcore_axis_name="s")

    @functools.partial(
        pl.kernel,
        out_type=[
            jax.ShapeDtypeStruct((NB, B), jnp.int32),
            jax.ShapeDtypeStruct((NC, NPAD, CW), jnp.float32),
            jax.ShapeDtypeStruct((NC, 128, CW), jnp.float32),
            jax.ShapeDtypeStruct((NC, 128, CW), jnp.float32),
        ],
        mesh=mesh,
        scratch_types=[
            pltpu.VMEM((max(base, 1), B), jnp.int32),
            pltpu.VMEM((max(base, 1), B), jnp.int32),
            pltpu.VMEM((gp, B), jnp.int32),
            pltpu.VMEM((B,), jnp.int32),
            pltpu.VMEM((B,), jnp.int32),
            pltpu.VMEM((B2,), jnp.int32),
            pltpu.VMEM((B, CW), jnp.float32),
            pltpu.SemaphoreType.DMA,
            pltpu.SemaphoreType.DMA,
            pltpu.VMEM_SHARED((NPAD, CW), jnp.float32),
            pltpu.VMEM_SHARED((128, CW), jnp.float32),
            pltpu.VMEM_SHARED((128, CW), jnp.float32),
        ],
        compiler_params=pltpu.CompilerParams(use_tc_tiling_on_sc=False),
    )
    def body(row_hbm, col_hbm, batch_hbm, zeros_hbm,
             birow_hbm, ccol_hbm, cb_hbm, cbr_hbm,
             idxar_v, idxac_v, bir_v, idxr_v, idxc_v, idxb_v, ones_v,
             semg, sems, ccol_s, cb_s, cbr_s):
        cid = lax.axis_index("c")
        sid = lax.axis_index("s")
        wid = sid * NC + cid

        def fill(r, c):
            ones_v[r, pl.ds(0, CW)] = jnp.full((CW,), 1.0, jnp.float32)
            return c

        lax.fori_loop(0, B, fill, 0)
        rpt = NPAD // NS
        pltpu.sync_copy(zeros_hbm.at[pl.ds(0, rpt)],
                        ccol_s.at[pl.ds(sid * rpt, rpt)])
        pltpu.sync_copy(zeros_hbm.at[pl.ds(0, 8)],
                        cb_s.at[pl.ds(sid * 8, 8)])
        pltpu.sync_copy(zeros_hbm.at[pl.ds(0, 8)],
                        cbr_s.at[pl.ds(sid * 8, 8)])
        plsc.subcore_barrier()

        start = wid * base + jnp.minimum(wid, rem)
        if base:
            pltpu.sync_copy(row_hbm.at[pl.ds(start, base)], idxar_v)
            pltpu.sync_copy(col_hbm.at[pl.ds(start, base)], idxac_v)

        def group(i, carry):
            descs = [pltpu.async_copy(batch_hbm.at[idxar_v.at[i * gp + j]],
                                      bir_v.at[j], semg)
                     for j in range(gp)]
            for d in descs:
                d.wait()
            pltpu.sync_copy(bir_v, birow_hbm.at[pl.ds(start + i * gp, gp)])
            descs = []
            for j in range(gp):
                lb = i * gp + j
                descs.append(pltpu.async_copy(
                    ones_v, ccol_s.at[idxac_v.at[lb]], sems, add=True))
                descs.append(pltpu.async_copy(
                    ones_v, cbr_s.at[bir_v.at[j]], sems, add=True))
            for d in descs:
                d.wait()
            return carry

        if nr:
            lax.fori_loop(0, nr, group, 0)
        for j in range(left):
            lb = nr * gp + j
            pltpu.async_copy(batch_hbm.at[idxar_v.at[lb]], bir_v.at[0],
                             semg).wait()
            pltpu.sync_copy(bir_v.at[pl.ds(0, 1)],
                            birow_hbm.at[pl.ds(start + lb, 1)])
            pltpu.sync_copy(ones_v, ccol_s.at[idxac_v.at[lb]], add=True)
            pltpu.sync_copy(ones_v, cbr_s.at[bir_v.at[0]], add=True)
        if rem:
            @pl.when(wid < rem)
            def _():
                b = start + base
                pltpu.sync_copy(row_hbm.at[b], idxr_v)
                pltpu.sync_copy(col_hbm.at[b], idxc_v)
                pltpu.async_copy(batch_hbm.at[idxr_v], bir_v.at[0],
                                 semg).wait()
                pltpu.sync_copy(bir_v.at[pl.ds(0, 1)],
                                birow_hbm.at[pl.ds(b, 1)])
                pltpu.sync_copy(ones_v, ccol_s.at[idxc_v], add=True)
                pltpu.sync_copy(ones_v, cbr_s.at[bir_v.at[0]], add=True)

        start2 = wid * base2 + jnp.minimum(wid, rem2)
        nb2 = base2 + jnp.where(wid < rem2, 1, 0)

        def blk2(i, carry):
            b = start2 + i
            pltpu.sync_copy(batch_hbm.at[pl.ds(b * B2, B2)], idxb_v)
            pltpu.sync_copy(ones_v.at[pl.ds(0, B2)], cb_s.at[idxb_v],
                            add=True)
            return carry

        lax.fori_loop(0, nb2, blk2, 0)
        plsc.subcore_barrier()
        pltpu.sync_copy(ccol_s.at[pl.ds(sid * rpt, rpt)],
                        ccol_hbm.at[cid, pl.ds(sid * rpt, rpt)])
        pltpu.sync_copy(cb_s.at[pl.ds(sid * 8, 8)],
                        cb_hbm.at[cid, pl.ds(sid * 8, 8)])
        pltpu.sync_copy(cbr_s.at[pl.ds(sid * 8, 8)],
                        cbr_hbm.at[cid, pl.ds(sid * 8, 8)])

    zeros = jnp.zeros((NPAD // NS, CW), jnp.float32)
    return body(row2d, col2d, batch, zeros)


def _pad_cols(w, width):
    return jnp.pad(w, ((0, 0), (0, width - w.shape[1])))


def _split_in(W, dims):
    out, o = [], 0
    for d in dims:
        out.append(W[o:o + d])
        o += d
    return out


def _meta_layer(p, x, ea, u, row2d, col2d, birow2d, OH,
                cnt_col, cnt_b, cnt_br,
                e_pad, x_pad, g_pad, ea_prev=None, xs=None, us=None):
    N, nf = x.shape
    E, ef = ea.shape
    NG, gf = u.shape

    e_dim_real = p["edge"]["lin_out"]["W"].shape[1]
    We, Wr, Wc, Wu = _split_in(p["edge"]["lin_in"]["W"], (ef, nf, nf, gf))
    Ax, Aa, Au = _split_in(p["node"]["lin_in"]["W"], (nf, e_dim_real, gf))
    Aa = jnp.pad(Aa, ((0, e_pad - e_dim_real), (0, 0)))
    Gu = p["global"]["lin_in"]["W"][:gf]

    Wut = jnp.concatenate([Wu, Au], axis=1)
    but = jnp.concatenate([p["edge"]["lin_in"]["b"],
                           p["node"]["lin_in"]["b"]])[None, :]
    T = _tmat(u, Wut, but)
    XRU, XC = _proj(x, jnp.concatenate([Wr, Wc], axis=1), OH, T)
    G0, G1 = _sc_gather([XRU, XC], [row2d, col2d], H)

    eW_hid = p["edge"]["lin_hid"]
    eW_out = p["edge"]["lin_out"]
    e_out = _edge_mlp(
        ea, G0, G1, We, eW_hid["W"], eW_hid["b"][None, :],
        _pad_cols(eW_out["W"], e_pad), _pad_cols(eW_out["b"][None, :], e_pad),
        ea_prev=ea_prev)

    ns0, ns1, ge0, ge1 = _scatter_edges(e_out, col2d, birow2d, N, NG)

    nW_hid = p["node"]["lin_hid"]
    nW_out = p["node"]["lin_out"]
    x_fresh, x_res, gx = _node_mlp(
        x, ns0, ns1, cnt_col[0], cnt_col[1], OH, T, Ax, Aa,
        nW_hid["W"], nW_hid["b"][None, :],
        _pad_cols(nW_out["W"], x_pad), _pad_cols(nW_out["b"][None, :], x_pad),
        xs=xs)

    Wg = p["global"]["lin_in"]["W"]
    e_dim_real = p["edge"]["lin_out"]["W"].shape[1]
    x_dim_real = Wg.shape[0] - gf - e_dim_real
    Gx_r = Wg[gf:gf + x_dim_real]
    Ge_r = Wg[gf + x_dim_real:]
    Gx_pad = jnp.pad(Gx_r, ((0, x_pad - Gx_r.shape[0]), (0, 0)))
    Ge_pad = jnp.pad(Ge_r, ((0, e_pad - Ge_r.shape[0]), (0, 0)))

    gW_hid = p["global"]["lin_hid"]
    gW_out = p["global"]["lin_out"]
    u_out = _global_mlp(
        u, gx, ge0, ge1, cnt_b, cnt_br, Gu, Gx_pad, Ge_pad,
        p["global"]["lin_in"]["b"][None, :], gW_hid["W"], gW_hid["b"][None, :],
        _pad_cols(gW_out["W"], g_pad), _pad_cols(gW_out["b"][None, :], g_pad),
        us=us)

    return x_fresh, x_res, e_out, u_out


def kernel(x, u, edge_index, edge_attr, batch, params):
    N, _ = x.shape
    NG, _ = u.shape
    E = edge_attr.shape[0]
    row, col = edge_index[0], edge_index[1]

    row2d = row.reshape(E // 128, 128)
    col2d = col.reshape(E // 128, 128)
    batch2d = batch.reshape(N // 80, 80)
    OH = _onehot(batch.reshape(N, 1), NG)
    birow2d, ccol_p, cb_p, cbr_p = _sc_precompute(
        row2d, col2d, batch, batch2d, N, NG)
    cnt_col = (ccol_p[0], ccol_p[1])
    cnt_b = (cb_p[0], cb_p[1])
    cnt_br = (cbr_p[0], cbr_p[1])

    pf, pm, plast = params["first"], params["mid"][0], params["last"]

    x1, _, e1, u1 = _meta_layer(
        pf, x, edge_attr, u, row2d, col2d, birow2d, OH,
        cnt_col, cnt_b, cnt_br, e_pad=H, x_pad=H, g_pad=H)

    x2f, x2, e2, u2 = _meta_layer(
        pm, x1, e1, u1, row2d, col2d, birow2d, OH,
        cnt_col, cnt_b, cnt_br, e_pad=H, x_pad=H, g_pad=H, xs=x1, us=u1)

    x3, _, e3, u3 = _meta_layer(
        plast, x2, e2, u2, row2d, col2d, birow2d, OH,
        cnt_col, cnt_b, cnt_br, e_pad=16, x_pad=16, g_pad=8, ea_prev=e1)

    return (x3[:, :7], e3[:, :6], u3[:, :1])
